# baseline (speedup 1.0000x reference)
import math

import jax
import jax.numpy as jnp
import numpy as np
from jax import lax
from jax.experimental import pallas as pl
from jax.experimental.pallas import tpu as pltpu
from jax.experimental.pallas import tpu_sc as plsc

NN = 10000
EE = 320000
HID = 64
NL = 4
OUTD = 20
CW = 128

NC, NS = 2, 16
NW = NC * NS
GB = 128
NBLK = EE // GB
WBLK = NBLK // NW
REM = NBLK - WBLK * NW

GPB = 2
GE = GB * GPB
NGRP = EE // GE
BASE_G = NGRP // NW
EXTRA_G = NGRP - BASE_G * NW

BE = 2000
NEB = EE // BE


def _silu(x):
    return x / (1.0 + jnp.exp(-x))


def _gelu_exact(x):
    return 0.5 * x * (1.0 + lax.erf(x * np.float32(1.0 / math.sqrt(2.0))))


def _sc_mesh():
    return plsc.VectorSubcoreMesh(
        core_axis_name="c", subcore_axis_name="s",
        num_cores=NC, num_subcores=NS)


def _worker_range(wid):
    ng = BASE_G + jnp.where(wid < EXTRA_G, 1, 0)
    g0 = wid * BASE_G + jnp.minimum(wid, EXTRA_G)
    return g0, ng


def _gather_body(tab_hbm, di_hbm, si_hbm,
                 gd_hbm, gs_hbm,
                 didx, sidx, gdb, gsb,
                 semi0, semi1, semg0, semg1, semw0, semw1):
    cid = lax.axis_index("c")
    sid = lax.axis_index("s")
    wid = sid * NC + cid
    g0 = wid * WBLK
    g_last = g0 + WBLK - 1
    semi = (semi0, semi1)
    semg = (semg0, semg1)
    semw = (semw0, semw1)

    def fire_idx(g, s, sem):
        pltpu.async_copy(di_hbm.at[g], didx.at[s], sem)
        pltpu.async_copy(si_hbm.at[g], sidx.at[s], sem)

    def drain_idx(s, sem):
        pltpu.make_async_copy(di_hbm.at[0], didx.at[s], sem).wait()
        pltpu.make_async_copy(si_hbm.at[0], sidx.at[s], sem).wait()

    def fire_gather(s, sem):
        pltpu.async_copy(tab_hbm.at[didx.at[s]], gdb.at[s], sem)
        pltpu.async_copy(tab_hbm.at[sidx.at[s]], gsb.at[s], sem)

    def drain_gather(s, sem):
        pltpu.make_async_copy(gd_hbm.at[pl.ds(0, GB)], gdb.at[s], sem).wait()
        pltpu.make_async_copy(gd_hbm.at[pl.ds(0, GB)], gsb.at[s], sem).wait()

    def fire_write(g, s, sem):
        eb = g * GB
        pltpu.async_copy(gdb.at[s], gd_hbm.at[pl.ds(eb, GB)], sem)
        pltpu.async_copy(gsb.at[s], gs_hbm.at[pl.ds(eb, GB)], sem)

    def drain_write(s, sem):
        pltpu.make_async_copy(gdb.at[s], gd_hbm.at[pl.ds(0, GB)], sem).wait()
        pltpu.make_async_copy(gsb.at[s], gs_hbm.at[pl.ds(0, GB)], sem).wait()

    fire_idx(g0, 0, semi[0])
    fire_idx(g0 + 1, 1, semi[1])

    def body(j, carry):
        for k in range(2):
            g = g0 + j * 2 + k
            o = 1 - k

            @pl.when(g - 1 >= g0)
            def _():
                drain_gather(o, semg[o])
                fire_write(g - 1, o, semw[o])

            @pl.when((g - 1 >= g0) & (g + 1 <= g_last))
            def _():
                fire_idx(g + 1, o, semi[o])

            @pl.when(g - 2 >= g0)
            def _():
                drain_write(k, semw[k])

            drain_idx(k, semi[k])
            fire_gather(k, semg[k])
        return carry

    lax.fori_loop(0, WBLK // 2, body, 0)

    drain_gather(1, semg[1])
    fire_write(g_last, 1, semw[1])
    drain_write(0, semw[0])
    drain_write(1, semw[1])

    @pl.when(wid < REM)
    def _():
        g = NW * WBLK + wid
        pltpu.sync_copy(di_hbm.at[g], didx.at[0])
        pltpu.sync_copy(si_hbm.at[g], sidx.at[0])
        pltpu.sync_copy(tab_hbm.at[didx.at[0]], gdb.at[0])
        pltpu.sync_copy(tab_hbm.at[sidx.at[0]], gsb.at[0])
        pltpu.sync_copy(gdb.at[0], gd_hbm.at[pl.ds(g * GB, GB)])
        pltpu.sync_copy(gsb.at[0], gs_hbm.at[pl.ds(g * GB, GB)])


_gather_call = pl.kernel(
    _gather_body,
    out_type=[
        jax.ShapeDtypeStruct((EE, CW), jnp.float32),
        jax.ShapeDtypeStruct((EE, CW), jnp.float32),
    ],
    mesh=_sc_mesh(),
    scratch_types=[
        pltpu.VMEM((2, GB), jnp.int32),
        pltpu.VMEM((2, GB), jnp.int32),
        pltpu.VMEM((2, GB, CW), jnp.float32),
        pltpu.VMEM((2, GB, CW), jnp.float32),
        pltpu.SemaphoreType.DMA,
        pltpu.SemaphoreType.DMA,
        pltpu.SemaphoreType.DMA,
        pltpu.SemaphoreType.DMA,
        pltpu.SemaphoreType.DMA,
        pltpu.SemaphoreType.DMA,
    ],
)


def _scatter_body(msg_hbm, di_hbm, z_hbm, acc_hbm, didx, mb, sh):
    cid = lax.axis_index("c")
    sid = lax.axis_index("s")
    wid = sid * NC + cid
    g0, ng = _worker_range(wid)

    @pl.when(sid == 0)
    def _():
        pltpu.sync_copy(z_hbm, sh)

    plsc.subcore_barrier()

    def body(i, carry):
        g = g0 + i
        eb = g * GE
        pltpu.sync_copy(di_hbm.at[pl.ds(g * GPB, GPB)], didx)
        pltpu.sync_copy(msg_hbm.at[pl.ds(eb, GE)], mb)
        for k in range(GPB):
            r = pl.ds(k * GB, GB)
            pltpu.sync_copy(mb.at[r], sh.at[didx.at[k]], add=True)
        return carry

    lax.fori_loop(0, ng, body, 0)
    plsc.subcore_barrier()

    @pl.when(sid < 10)
    def _():
        r = pl.ds(sid * 1000, 1000)
        pltpu.sync_copy(sh.at[r], acc_hbm.at[cid, r])


_scatter_call = pl.kernel(
    _scatter_body,
    out_type=[jax.ShapeDtypeStruct((NC, NN, CW), jnp.float32)],
    mesh=_sc_mesh(),
    scratch_types=[
        pltpu.VMEM((GPB, GB), jnp.int32),
        pltpu.VMEM((GE, CW), jnp.float32),
        pltpu.VMEM_SHARED((NN, CW), jnp.float32),
    ],
)


def _unpack_row(gc):
    return gc[:, :HID], gc[:, HID:HID + 3]


def _pos_rel(pd, ps):
    rel3 = pd - ps
    rd = jnp.sum(rel3 * rel3, axis=1, keepdims=True)
    return rel3, rd


def _edge_tc_body(gdc, gsc, ea,
                  w1a, w1b, w1c, w1d, b1, w2, b2,
                  wu, bu, wc1, bc1, wc2, bc2,
                  msg_o, eo_o):
    gd, pd = _unpack_row(gdc[...])
    gs, ps = _unpack_row(gsc[...])
    eaf = ea[...].astype(jnp.float32)
    rel3, rd = _pos_rel(pd, ps)
    x1 = (gd @ w1a[...] + gs @ w1b[...] + eaf @ w1d[...]
          + rd * w1c[...] + b1[...])
    h1 = _silu(x1)
    mm = _silu(h1 @ w2[...] + b2[...])
    eo_o[...] = (mm @ wu[...] + bu[...] + eaf).astype(eo_o.dtype)
    c1 = _silu(mm @ wc1[...] + bc1[...])
    cw = jnp.sum(c1 * wc2[...], axis=1, keepdims=True) + bc2[...]
    pad = jnp.zeros((BE, CW - HID - 3), jnp.float32)
    msg_o[...] = jnp.concatenate([mm, rel3 * cw, pad], axis=1)


def _edge_tc_body_last(gdc, gsc, ea,
                       w1a, w1b, w1c, w1d, b1, w2, b2,
                       msg_o):
    gd, pd = _unpack_row(gdc[...])
    gs, ps = _unpack_row(gsc[...])
    eaf = ea[...].astype(jnp.float32)
    rel3, rd = _pos_rel(pd, ps)
    x1 = (gd @ w1a[...] + gs @ w1b[...] + eaf @ w1d[...]
          + rd * w1c[...] + b1[...])
    mm = _silu(_silu(x1) @ w2[...] + b2[...])
    pad = jnp.zeros((BE, CW - HID), jnp.float32)
    msg_o[...] = jnp.concatenate([mm, pad], axis=1)


def _eb(d):
    return pl.BlockSpec((BE, d), lambda i: (i, 0))


def _wb(shape):
    nd = len(shape)
    return pl.BlockSpec(shape, lambda i: (0,) * nd)


_EDGE_W_SPECS = [
    _wb((HID, HID)), _wb((HID, HID)), _wb((1, HID)), _wb((HID, HID)),
    _wb((1, HID)), _wb((HID, HID)), _wb((1, HID)),
]


_edge_call = pl.pallas_call(
    _edge_tc_body,
    grid=(NEB,),
    in_specs=[_eb(CW), _eb(CW), _eb(HID)]
             + _EDGE_W_SPECS
             + [_wb((HID, HID)), _wb((1, HID)), _wb((HID, HID)),
                _wb((1, HID)), _wb((1, HID)), _wb((1, 1))],
    out_specs=[_eb(CW), _eb(HID)],
    out_shape=[
        jax.ShapeDtypeStruct((EE, CW), jnp.float32),
        jax.ShapeDtypeStruct((EE, HID), jnp.bfloat16),
    ],
)

_edge_last_call = pl.pallas_call(
    _edge_tc_body_last,
    grid=(NEB,),
    in_specs=[_eb(CW), _eb(CW), _eb(HID)] + _EDGE_W_SPECS,
    out_specs=[_eb(CW)],
    out_shape=[jax.ShapeDtypeStruct((EE, CW), jnp.float32)],
)


_INV_CNT = np.float32(1.0 / (NN * HID))


def _node_core(tab, acc, sc, sh, wn1a, wn1b, bn1, wn2, bn2,
               g_, be_, wf1, bf1, wf2, bf2):
    f0 = tab[:, :HID]
    m_i = acc[0][:, :HID] + acc[1][:, :HID]
    nh = _silu(f0 @ wn1a[...] + m_i @ wn1b[...] + bn1[...])
    nh = nh @ wn2[...] + bn2[...] + f0
    f = nh * (sc[...] + 1.0) + sh[...]
    mean = jnp.sum(f) * _INV_CNT
    xc = f - mean
    var = jnp.sum(xc * xc) * _INV_CNT
    fn = xc * lax.rsqrt(var + np.float32(1e-5)) * g_[...] + be_[...]
    fh = _gelu_exact(fn @ wf1[...] + bf1[...])
    return fh @ wf2[...] + bf2[...] + fn


def _node_tc_body(tab_r, acc_r, sc, sh,
                  wn1a, wn1b, bn1, wn2, bn2, g_, be_, wf1, bf1, wf2, bf2,
                  tab_o):
    tab = tab_r[...]
    acc = acc_r[...]
    fnew = _node_core(tab, acc, sc, sh, wn1a, wn1b, bn1, wn2, bn2,
                      g_, be_, wf1, bf1, wf2, bf2)
    pos = tab[:, HID:HID + 16] + acc[0][:, HID:HID + 16] + acc[1][:, HID:HID + 16]
    pad = jnp.zeros((NN, CW - HID - 16), jnp.float32)
    tab_o[...] = jnp.concatenate([fnew, pos, pad], axis=1)


def _node_tc_body_last(tab_r, acc_r, sc, sh,
                       wn1a, wn1b, bn1, wn2, bn2, g_, be_, wf1, bf1, wf2, bf2,
                       wlin, blin, out_o):
    f = _node_core(tab_r[...], acc_r[...], sc, sh, wn1a, wn1b, bn1, wn2, bn2,
                   g_, be_, wf1, bf1, wf2, bf2)
    out_o[...] = f @ wlin[...] + blin[...]


_node_call = pl.pallas_call(
    _node_tc_body,
    out_shape=[jax.ShapeDtypeStruct((NN, CW), jnp.float32)],
)

_node_last_call = pl.pallas_call(
    _node_tc_body_last,
    out_shape=[jax.ShapeDtypeStruct((NN, OUTD), jnp.float32)],
)


def _time_tc_body(tval, freqs, wtm1, btm1, wtm2, btm2, wt, bt, temb_o):
    e = tval[...] * freqs[...]
    emb = jnp.concatenate([jnp.sin(e), jnp.cos(e)], axis=1)
    t1 = _silu(emb @ wtm1[...] + btm1[...])
    t2 = t1 @ wtm2[...] + btm2[...]
    st = _silu(t2)
    temb_o[...] = st @ wt[...] + bt[...]


_time_call = pl.pallas_call(
    _time_tc_body,
    out_shape=[jax.ShapeDtypeStruct((1, 2 * HID * NL), jnp.float32)],
)


def _row(b):
    return b.reshape(1, -1)


def kernel(x, pos, extra_x, edge_attr, ss, time, params, edge_index, batch):
    del ss, batch
    tab = jnp.concatenate(
        [x, extra_x, pos, jnp.zeros((NN, CW - 2 * 32 - 3), jnp.float32)], axis=1)
    si2d = edge_index[0].reshape(NGRP * GPB, GB)
    di2d = edge_index[1].reshape(NGRP * GPB, GB)
    zc = jnp.zeros((NN, CW), jnp.float32)

    half = HID // 2
    freqs = jnp.exp(
        jnp.arange(half, dtype=jnp.float32)
        * np.float32(-math.log(10000.0) / (half - 1))).reshape(1, half)
    tm1, tm2 = params["time_mlp"]
    wt = jnp.concatenate([l["time"]["w"] for l in params["layers"]], axis=1)
    bt = jnp.concatenate([l["time"]["b"] for l in params["layers"]]).reshape(1, -1)
    (temb,) = _time_call(time.reshape(1, 1), freqs,
                         tm1["w"], _row(tm1["b"]), tm2["w"], _row(tm2["b"]),
                         wt, bt)

    ea = edge_attr
    out = None
    for l, lay in enumerate(params["layers"]):
        gd, gs = _gather_call(tab, di2d, si2d)

        w1 = lay["edge_mlp"][0]["w"]
        ew = (w1[:HID], w1[HID:2 * HID], w1[2 * HID:2 * HID + 1],
              w1[2 * HID + 1:], _row(lay["edge_mlp"][0]["b"]),
              lay["edge_mlp"][1]["w"], _row(lay["edge_mlp"][1]["b"]))
        sc = temb[:, 2 * HID * l: 2 * HID * l + HID]
        sh = temb[:, 2 * HID * l + HID: 2 * HID * (l + 1)]
        nw1 = lay["node_mlp"][0]["w"]
        nws = (nw1[:HID], nw1[HID:], _row(lay["node_mlp"][0]["b"]),
               lay["node_mlp"][1]["w"], _row(lay["node_mlp"][1]["b"]),
               _row(lay["ff_norm"]["g"]), _row(lay["ff_norm"]["be"]),
               lay["ff"][0]["w"], _row(lay["ff"][0]["b"]),
               lay["ff"][1]["w"], _row(lay["ff"][1]["b"]))

        if l < NL - 1:
            msg, eout = _edge_call(
                gd, gs, ea, *ew,
                lay["edge_upd"]["w"], _row(lay["edge_upd"]["b"]),
                lay["coors_mlp"][0]["w"], _row(lay["coors_mlp"][0]["b"]),
                lay["coors_mlp"][1]["w"].reshape(1, HID),
                lay["coors_mlp"][1]["b"].reshape(1, 1))
            (acc,) = _scatter_call(msg, di2d, zc)
            (tab,) = _node_call(tab, acc, sc, sh, *nws)
            ea = eout
        else:
            (msg,) = _edge_last_call(gd, gs, ea, *ew)
            (acc,) = _scatter_call(msg, di2d, zc)
            (out,) = _node_last_call(tab, acc, sc, sh, *nws,
                                     params["lin"]["w"],
                                     _row(params["lin"]["b"]))
    return out


# speedup vs baseline: 1.1584x; 1.1584x over previous
import math

import jax
import jax.numpy as jnp
import numpy as np
from jax import lax
from jax.experimental import pallas as pl
from jax.experimental.pallas import tpu as pltpu
from jax.experimental.pallas import tpu_sc as plsc

NN = 10000
EE = 320000
HH = EE // 2
HID = 64
NL = 4
OUTD = 20
CW = 128

NC, NS = 2, 16
NW = NC * NS
GB = 128

GPB = 2
GE = GB * GPB

BE = 2000


def _silu(x):
    return x / (1.0 + jnp.exp(-x))


def _gelu_exact(x):
    return 0.5 * x * (1.0 + lax.erf(x * np.float32(1.0 / math.sqrt(2.0))))


def _sc_mesh():
    return plsc.VectorSubcoreMesh(
        core_axis_name="c", subcore_axis_name="s",
        num_cores=NC, num_subcores=NS)


def _partition(wid, base, extra):
    ng = base + jnp.where(wid < extra, 1, 0)
    g0 = wid * base + jnp.minimum(wid, extra)
    return g0, ng


def _make_gather(ne):
    nblk = ne // GB
    base, extra = nblk // NW, nblk % NW

    def body(tab_hbm, di_hbm, si_hbm, gd_hbm, gs_hbm,
             didx, sidx, gdb, gsb,
             semi0, semi1, semg0, semg1, semw0, semw1):
        cid = lax.axis_index("c")
        sid = lax.axis_index("s")
        wid = sid * NC + cid
        g0, ng = _partition(wid, base, extra)
        nmain = (ng // 2) * 2
        g_last = g0 + nmain - 1
        semi = (semi0, semi1)
        semg = (semg0, semg1)
        semw = (semw0, semw1)

        def fire_idx(g, s, sem):
            pltpu.async_copy(di_hbm.at[g], didx.at[s], sem)
            pltpu.async_copy(si_hbm.at[g], sidx.at[s], sem)

        def drain_idx(s, sem):
            pltpu.make_async_copy(di_hbm.at[0], didx.at[s], sem).wait()
            pltpu.make_async_copy(si_hbm.at[0], sidx.at[s], sem).wait()

        def fire_gather(s, sem):
            pltpu.async_copy(tab_hbm.at[didx.at[s]], gdb.at[s], sem)
            pltpu.async_copy(tab_hbm.at[sidx.at[s]], gsb.at[s], sem)

        def drain_gather(s, sem):
            pltpu.make_async_copy(gd_hbm.at[pl.ds(0, GB)], gdb.at[s], sem).wait()
            pltpu.make_async_copy(gd_hbm.at[pl.ds(0, GB)], gsb.at[s], sem).wait()

        def fire_write(g, s, sem):
            eb = g * GB
            pltpu.async_copy(gdb.at[s], gd_hbm.at[pl.ds(eb, GB)], sem)
            pltpu.async_copy(gsb.at[s], gs_hbm.at[pl.ds(eb, GB)], sem)

        def drain_write(s, sem):
            pltpu.make_async_copy(gdb.at[s], gd_hbm.at[pl.ds(0, GB)], sem).wait()
            pltpu.make_async_copy(gsb.at[s], gs_hbm.at[pl.ds(0, GB)], sem).wait()

        fire_idx(g0, 0, semi[0])
        fire_idx(g0 + 1, 1, semi[1])

        def loop(j, carry):
            for k in range(2):
                g = g0 + j * 2 + k
                o = 1 - k

                @pl.when(g - 1 >= g0)
                def _():
                    drain_gather(o, semg[o])
                    fire_write(g - 1, o, semw[o])

                @pl.when((g - 1 >= g0) & (g + 1 <= g_last))
                def _():
                    fire_idx(g + 1, o, semi[o])

                @pl.when(g - 2 >= g0)
                def _():
                    drain_write(k, semw[k])

                drain_idx(k, semi[k])
                fire_gather(k, semg[k])
            return carry

        lax.fori_loop(0, ng // 2, loop, 0)

        drain_gather(1, semg[1])
        fire_write(g_last, 1, semw[1])
        drain_write(0, semw[0])
        drain_write(1, semw[1])

        @pl.when(ng > nmain)
        def _():
            g = g0 + nmain
            pltpu.sync_copy(di_hbm.at[g], didx.at[0])
            pltpu.sync_copy(si_hbm.at[g], sidx.at[0])
            pltpu.sync_copy(tab_hbm.at[didx.at[0]], gdb.at[0])
            pltpu.sync_copy(tab_hbm.at[sidx.at[0]], gsb.at[0])
            pltpu.sync_copy(gdb.at[0], gd_hbm.at[pl.ds(g * GB, GB)])
            pltpu.sync_copy(gsb.at[0], gs_hbm.at[pl.ds(g * GB, GB)])

    return pl.kernel(
        body,
        out_type=[
            jax.ShapeDtypeStruct((ne, CW), jnp.float32),
            jax.ShapeDtypeStruct((ne, CW), jnp.float32),
        ],
        mesh=_sc_mesh(),
        scratch_types=[
            pltpu.VMEM((2, GB), jnp.int32),
            pltpu.VMEM((2, GB), jnp.int32),
            pltpu.VMEM((2, GB, CW), jnp.float32),
            pltpu.VMEM((2, GB, CW), jnp.float32),
            pltpu.SemaphoreType.DMA,
            pltpu.SemaphoreType.DMA,
            pltpu.SemaphoreType.DMA,
            pltpu.SemaphoreType.DMA,
            pltpu.SemaphoreType.DMA,
            pltpu.SemaphoreType.DMA,
        ],
    )


_gather_h = _make_gather(HH)


def _make_scatter(ne):
    ngrp = ne // GE
    base, extra = ngrp // NW, ngrp % NW

    def body(msg_hbm, di_hbm, z_hbm, acc_hbm, didx, mb, sh):
        cid = lax.axis_index("c")
        sid = lax.axis_index("s")
        wid = sid * NC + cid
        g0, ng = _partition(wid, base, extra)

        @pl.when(sid == 0)
        def _():
            pltpu.sync_copy(z_hbm, sh)

        plsc.subcore_barrier()

        def loop(i, carry):
            g = g0 + i
            eb = g * GE
            pltpu.sync_copy(di_hbm.at[pl.ds(g * GPB, GPB)], didx)
            pltpu.sync_copy(msg_hbm.at[pl.ds(eb, GE)], mb)
            for k in range(GPB):
                r = pl.ds(k * GB, GB)
                pltpu.sync_copy(mb.at[r], sh.at[didx.at[k]], add=True)
            return carry

        lax.fori_loop(0, ng, loop, 0)
        plsc.subcore_barrier()

        @pl.when(sid < 10)
        def _():
            r = pl.ds(sid * 1000, 1000)
            pltpu.sync_copy(sh.at[r], acc_hbm.at[cid, r])

    return pl.kernel(
        body,
        out_type=[jax.ShapeDtypeStruct((NC, NN, CW), jnp.float32)],
        mesh=_sc_mesh(),
        scratch_types=[
            pltpu.VMEM((GPB, GB), jnp.int32),
            pltpu.VMEM((GE, CW), jnp.float32),
            pltpu.VMEM_SHARED((NN, CW), jnp.float32),
        ],
    )


_scatter_h = _make_scatter(HH)


def _edge_tc_body(gdc, gsc, ea,
                  w1a, w1b, w1c, w1d, b1, w2, b2,
                  wu, bu, wc1, bc1, wc2, bc2,
                  msg_o, eo_o):
    gd = gdc[:, :HID]
    gs = gsc[:, :HID]
    rel3 = gdc[:, HID:HID + 3] - gsc[:, HID:HID + 3]
    rd = jnp.sum(rel3 * rel3, axis=1, keepdims=True)
    eaf = ea[...].astype(jnp.float32)
    x1 = (gd @ w1a[...] + gs @ w1b[...] + eaf @ w1d[...]
          + rd * w1c[...] + b1[...])
    h1 = _silu(x1)
    mm = _silu(h1 @ w2[...] + b2[...])
    eo_o[...] = (mm @ wu[...] + bu[...] + eaf).astype(eo_o.dtype)
    c1 = _silu(mm @ wc1[...] + bc1[...])
    cw = jnp.sum(c1 * wc2[...], axis=1, keepdims=True) + bc2[...]
    pad = jnp.zeros((BE, CW - HID - 3), jnp.float32)
    msg_o[...] = jnp.concatenate([mm, rel3 * cw, pad], axis=1)


def _edge_tc_body_last(gdc, gsc, ea,
                       w1a, w1b, w1c, w1d, b1, w2, b2,
                       msg_o):
    gd = gdc[:, :HID]
    gs = gsc[:, :HID]
    rel3 = gdc[:, HID:HID + 3] - gsc[:, HID:HID + 3]
    rd = jnp.sum(rel3 * rel3, axis=1, keepdims=True)
    eaf = ea[...].astype(jnp.float32)
    x1 = (gd @ w1a[...] + gs @ w1b[...] + eaf @ w1d[...]
          + rd * w1c[...] + b1[...])
    mm = _silu(_silu(x1) @ w2[...] + b2[...])
    pad = jnp.zeros((BE, CW - HID), jnp.float32)
    msg_o[...] = jnp.concatenate([mm, pad], axis=1)


def _eb(d):
    return pl.BlockSpec((BE, d), lambda i: (i, 0))


def _wb(shape):
    nd = len(shape)
    return pl.BlockSpec(shape, lambda i: (0,) * nd)


_EDGE_W_SPECS = [
    _wb((HID, HID)), _wb((HID, HID)), _wb((1, HID)), _wb((HID, HID)),
    _wb((1, HID)), _wb((HID, HID)), _wb((1, HID)),
]

_edge_h = pl.pallas_call(
    _edge_tc_body,
    grid=(HH // BE,),
    in_specs=[_eb(CW), _eb(CW), _eb(HID)]
             + _EDGE_W_SPECS
             + [_wb((HID, HID)), _wb((1, HID)), _wb((HID, HID)),
                _wb((1, HID)), _wb((1, HID)), _wb((1, 1))],
    out_specs=[_eb(CW), _eb(HID)],
    out_shape=[
        jax.ShapeDtypeStruct((HH, CW), jnp.float32),
        jax.ShapeDtypeStruct((HH, HID), jnp.bfloat16),
    ],
)

_edge_last_h = pl.pallas_call(
    _edge_tc_body_last,
    grid=(HH // BE,),
    in_specs=[_eb(CW), _eb(CW), _eb(HID)] + _EDGE_W_SPECS,
    out_specs=[_eb(CW)],
    out_shape=[jax.ShapeDtypeStruct((HH, CW), jnp.float32)],
)


_INV_CNT = np.float32(1.0 / (NN * HID))


def _node_core(tab, msum, sc, sh, wn1a, wn1b, bn1, wn2, bn2,
               g_, be_, wf1, bf1, wf2, bf2):
    f0 = tab[:, :HID]
    m_i = msum[:, :HID]
    nh = _silu(f0 @ wn1a[...] + m_i @ wn1b[...] + bn1[...])
    nh = nh @ wn2[...] + bn2[...] + f0
    f = nh * (sc[...] + 1.0) + sh[...]
    mean = jnp.sum(f) * _INV_CNT
    xc = f - mean
    var = jnp.sum(xc * xc) * _INV_CNT
    fn = xc * lax.rsqrt(var + np.float32(1e-5)) * g_[...] + be_[...]
    fh = _gelu_exact(fn @ wf1[...] + bf1[...])
    return fh @ wf2[...] + bf2[...] + fn


def _node_tc_body(tab_r, acc0_r, acc1_r, sc, sh,
                  wn1a, wn1b, bn1, wn2, bn2, g_, be_, wf1, bf1, wf2, bf2,
                  tab_o):
    tab = tab_r[...]
    msum = acc0_r[0] + acc0_r[1] + acc1_r[0] + acc1_r[1]
    fnew = _node_core(tab, msum, sc, sh, wn1a, wn1b, bn1, wn2, bn2,
                      g_, be_, wf1, bf1, wf2, bf2)
    pos = tab[:, HID:HID + 16] + msum[:, HID:HID + 16]
    pad = jnp.zeros((NN, CW - HID - 16), jnp.float32)
    tab_o[...] = jnp.concatenate([fnew, pos, pad], axis=1)


def _node_tc_body_last(tab_r, acc0_r, acc1_r, sc, sh,
                       wn1a, wn1b, bn1, wn2, bn2, g_, be_, wf1, bf1, wf2, bf2,
                       wlin, blin, out_o):
    msum = acc0_r[0] + acc0_r[1] + acc1_r[0] + acc1_r[1]
    f = _node_core(tab_r[...], msum, sc, sh, wn1a, wn1b, bn1, wn2, bn2,
                   g_, be_, wf1, bf1, wf2, bf2)
    out_o[...] = f @ wlin[...] + blin[...]


_node_call = pl.pallas_call(
    _node_tc_body,
    out_shape=[jax.ShapeDtypeStruct((NN, CW), jnp.float32)],
)

_node_last_call = pl.pallas_call(
    _node_tc_body_last,
    out_shape=[jax.ShapeDtypeStruct((NN, OUTD), jnp.float32)],
)


def _time_tc_body(tval, freqs, wtm1, btm1, wtm2, btm2, wt, bt, temb_o):
    e = tval[...] * freqs[...]
    emb = jnp.concatenate([jnp.sin(e), jnp.cos(e)], axis=1)
    t1 = _silu(emb @ wtm1[...] + btm1[...])
    t2 = t1 @ wtm2[...] + btm2[...]
    st = _silu(t2)
    temb_o[...] = st @ wt[...] + bt[...]


_time_call = pl.pallas_call(
    _time_tc_body,
    out_shape=[jax.ShapeDtypeStruct((1, 2 * HID * NL), jnp.float32)],
)


def _row(b):
    return b.reshape(1, -1)


def kernel(x, pos, extra_x, edge_attr, ss, time, params, edge_index, batch):
    del ss, batch
    tab = jnp.concatenate(
        [x, extra_x, pos, jnp.zeros((NN, CW - 2 * 32 - 3), jnp.float32)], axis=1)
    si = [edge_index[0, h * HH:(h + 1) * HH].reshape(HH // GB, GB)
          for h in range(2)]
    di = [edge_index[1, h * HH:(h + 1) * HH].reshape(HH // GB, GB)
          for h in range(2)]
    ea = [edge_attr[:HH], edge_attr[HH:]]
    zc = jnp.zeros((NN, CW), jnp.float32)

    half = HID // 2
    freqs = jnp.exp(
        jnp.arange(half, dtype=jnp.float32)
        * np.float32(-math.log(10000.0) / (half - 1))).reshape(1, half)
    tm1, tm2 = params["time_mlp"]
    wt = jnp.concatenate([l["time"]["w"] for l in params["layers"]], axis=1)
    bt = jnp.concatenate([l["time"]["b"] for l in params["layers"]]).reshape(1, -1)
    (temb,) = _time_call(time.reshape(1, 1), freqs,
                         tm1["w"], _row(tm1["b"]), tm2["w"], _row(tm2["b"]),
                         wt, bt)

    out = None
    for l, lay in enumerate(params["layers"]):
        w1 = lay["edge_mlp"][0]["w"]
        ew = (w1[:HID], w1[HID:2 * HID], w1[2 * HID:2 * HID + 1],
              w1[2 * HID + 1:], _row(lay["edge_mlp"][0]["b"]),
              lay["edge_mlp"][1]["w"], _row(lay["edge_mlp"][1]["b"]))
        euw = (lay["edge_upd"]["w"], _row(lay["edge_upd"]["b"]),
               lay["coors_mlp"][0]["w"], _row(lay["coors_mlp"][0]["b"]),
               lay["coors_mlp"][1]["w"].reshape(1, HID),
               lay["coors_mlp"][1]["b"].reshape(1, 1))
        sc = temb[:, 2 * HID * l: 2 * HID * l + HID]
        sh = temb[:, 2 * HID * l + HID: 2 * HID * (l + 1)]
        nw1 = lay["node_mlp"][0]["w"]
        nws = (nw1[:HID], nw1[HID:], _row(lay["node_mlp"][0]["b"]),
               lay["node_mlp"][1]["w"], _row(lay["node_mlp"][1]["b"]),
               _row(lay["ff_norm"]["g"]), _row(lay["ff_norm"]["be"]),
               lay["ff"][0]["w"], _row(lay["ff"][0]["b"]),
               lay["ff"][1]["w"], _row(lay["ff"][1]["b"]))

        gd0, gs0 = _gather_h(tab, di[0], si[0])
        gd1, gs1 = _gather_h(tab, di[1], si[1])
        if l < NL - 1:
            msg0, eo0 = _edge_h(gd0, gs0, ea[0], *ew, *euw)
            (acc0,) = _scatter_h(msg0, di[0], zc)
            msg1, eo1 = _edge_h(gd1, gs1, ea[1], *ew, *euw)
            (acc1,) = _scatter_h(msg1, di[1], zc)
            (tab,) = _node_call(tab, acc0, acc1, sc, sh, *nws)
            ea = [eo0, eo1]
        else:
            (msg0,) = _edge_last_h(gd0, gs0, ea[0], *ew)
            (acc0,) = _scatter_h(msg0, di[0], zc)
            (msg1,) = _edge_last_h(gd1, gs1, ea[1], *ew)
            (acc1,) = _scatter_h(msg1, di[1], zc)
            (out,) = _node_last_call(tab, acc0, acc1, sc, sh, *nws,
                                     params["lin"]["w"],
                                     _row(params["lin"]["b"]))
    return out


# speedup vs baseline: 1.1861x; 1.0239x over previous
import math

import jax
import jax.numpy as jnp
import numpy as np
from jax import lax
from jax.experimental import pallas as pl
from jax.experimental.pallas import tpu as pltpu
from jax.experimental.pallas import tpu_sc as plsc

NN = 10000
EE = 320000
HH = EE // 2
HID = 64
NL = 4
OUTD = 20
CW = 128

NC, NS = 2, 16
NW = NC * NS
GB = 128

GPB = 1
GE = GB * GPB

BE = 2000


def _silu(x):
    return x / (1.0 + jnp.exp(-x))


def _gelu_exact(x):
    return 0.5 * x * (1.0 + lax.erf(x * np.float32(1.0 / math.sqrt(2.0))))


def _sc_mesh():
    return plsc.VectorSubcoreMesh(
        core_axis_name="c", subcore_axis_name="s",
        num_cores=NC, num_subcores=NS)


def _partition(wid, base, extra):
    ng = base + jnp.where(wid < extra, 1, 0)
    g0 = wid * base + jnp.minimum(wid, extra)
    return g0, ng


def _make_gather(ne):
    nblk = ne // GB
    base, extra = nblk // NW, nblk % NW

    def body(tab_hbm, di_hbm, si_hbm, gd_hbm, gs_hbm,
             didx, sidx, gdb, gsb,
             semi0, semi1, semg0, semg1, semw0, semw1):
        cid = lax.axis_index("c")
        sid = lax.axis_index("s")
        wid = sid * NC + cid
        g0, ng = _partition(wid, base, extra)
        nmain = (ng // 2) * 2
        g_last = g0 + nmain - 1
        semi = (semi0, semi1)
        semg = (semg0, semg1)
        semw = (semw0, semw1)

        def fire_idx(g, s, sem):
            pltpu.async_copy(di_hbm.at[g], didx.at[s], sem)
            pltpu.async_copy(si_hbm.at[g], sidx.at[s], sem)

        def drain_idx(s, sem):
            pltpu.make_async_copy(di_hbm.at[0], didx.at[s], sem).wait()
            pltpu.make_async_copy(si_hbm.at[0], sidx.at[s], sem).wait()

        def fire_gather(s, sem):
            pltpu.async_copy(tab_hbm.at[didx.at[s]], gdb.at[s], sem)
            pltpu.async_copy(tab_hbm.at[sidx.at[s]], gsb.at[s], sem)

        def drain_gather(s, sem):
            pltpu.make_async_copy(gd_hbm.at[pl.ds(0, GB)], gdb.at[s], sem).wait()
            pltpu.make_async_copy(gd_hbm.at[pl.ds(0, GB)], gsb.at[s], sem).wait()

        def fire_write(g, s, sem):
            eb = g * GB
            pltpu.async_copy(gdb.at[s], gd_hbm.at[pl.ds(eb, GB)], sem)
            pltpu.async_copy(gsb.at[s], gs_hbm.at[pl.ds(eb, GB)], sem)

        def drain_write(s, sem):
            pltpu.make_async_copy(gdb.at[s], gd_hbm.at[pl.ds(0, GB)], sem).wait()
            pltpu.make_async_copy(gsb.at[s], gs_hbm.at[pl.ds(0, GB)], sem).wait()

        fire_idx(g0, 0, semi[0])
        fire_idx(g0 + 1, 1, semi[1])

        def loop(j, carry):
            for k in range(2):
                g = g0 + j * 2 + k
                o = 1 - k

                @pl.when(g - 1 >= g0)
                def _():
                    drain_gather(o, semg[o])
                    fire_write(g - 1, o, semw[o])

                @pl.when((g - 1 >= g0) & (g + 1 <= g_last))
                def _():
                    fire_idx(g + 1, o, semi[o])

                @pl.when(g - 2 >= g0)
                def _():
                    drain_write(k, semw[k])

                drain_idx(k, semi[k])
                fire_gather(k, semg[k])
            return carry

        lax.fori_loop(0, ng // 2, loop, 0)

        drain_gather(1, semg[1])
        fire_write(g_last, 1, semw[1])
        drain_write(0, semw[0])
        drain_write(1, semw[1])

        @pl.when(ng > nmain)
        def _():
            g = g0 + nmain
            pltpu.sync_copy(di_hbm.at[g], didx.at[0])
            pltpu.sync_copy(si_hbm.at[g], sidx.at[0])
            pltpu.sync_copy(tab_hbm.at[didx.at[0]], gdb.at[0])
            pltpu.sync_copy(tab_hbm.at[sidx.at[0]], gsb.at[0])
            pltpu.sync_copy(gdb.at[0], gd_hbm.at[pl.ds(g * GB, GB)])
            pltpu.sync_copy(gsb.at[0], gs_hbm.at[pl.ds(g * GB, GB)])

    return pl.kernel(
        body,
        out_type=[
            jax.ShapeDtypeStruct((ne, CW), jnp.float32),
            jax.ShapeDtypeStruct((ne, CW), jnp.float32),
        ],
        mesh=_sc_mesh(),
        scratch_types=[
            pltpu.VMEM((2, GB), jnp.int32),
            pltpu.VMEM((2, GB), jnp.int32),
            pltpu.VMEM((2, GB, CW), jnp.float32),
            pltpu.VMEM((2, GB, CW), jnp.float32),
            pltpu.SemaphoreType.DMA,
            pltpu.SemaphoreType.DMA,
            pltpu.SemaphoreType.DMA,
            pltpu.SemaphoreType.DMA,
            pltpu.SemaphoreType.DMA,
            pltpu.SemaphoreType.DMA,
        ],
    )


_gather_h = _make_gather(HH)


def _make_scatter(ne):
    ngrp = ne // GE
    base, extra = ngrp // NW, ngrp % NW

    def body(msg_hbm, di_hbm, z_hbm, acc_hbm, didx, mb, sh,
             semp0, semp1, sema0, sema1):
        cid = lax.axis_index("c")
        sid = lax.axis_index("s")
        wid = sid * NC + cid
        g0, ng = _partition(wid, base, extra)
        nmain = (ng // 2) * 2
        g_last = g0 + nmain - 1
        semp = (semp0, semp1)
        sema = (sema0, sema1)

        @pl.when(sid == 0)
        def _():
            pltpu.sync_copy(z_hbm, sh)

        plsc.subcore_barrier()

        def fire_pf(g, s):
            pltpu.async_copy(di_hbm.at[pl.ds(g * GPB, GPB)], didx.at[s], semp[s])
            pltpu.async_copy(msg_hbm.at[pl.ds(g * GE, GE)], mb.at[s], semp[s])

        def drain_pf(s):
            pltpu.make_async_copy(
                di_hbm.at[pl.ds(0, GPB)], didx.at[s], semp[s]).wait()
            pltpu.make_async_copy(
                msg_hbm.at[pl.ds(0, GE)], mb.at[s], semp[s]).wait()

        def fire_adds(s):
            for k in range(GPB):
                pltpu.async_copy(mb.at[s, pl.ds(k * GB, GB)],
                                 sh.at[didx.at[s, k]], sema[s], add=True)

        def drain_adds(s):
            for k in range(GPB):
                pltpu.make_async_copy(mb.at[s, pl.ds(k * GB, GB)],
                                      sh.at[pl.ds(0, GB)], sema[s]).wait()

        fire_pf(g0, 0)

        def loop(j, carry):
            for k in range(2):
                g = g0 + j * 2 + k
                o = 1 - k
                drain_pf(k)

                @pl.when(g - 1 >= g0)
                def _():
                    drain_adds(o)

                @pl.when(g + 1 <= g_last)
                def _():
                    fire_pf(g + 1, o)

                fire_adds(k)
            return carry

        lax.fori_loop(0, ng // 2, loop, 0)
        drain_adds(1)

        @pl.when(ng > nmain)
        def _():
            g = g0 + nmain
            pltpu.sync_copy(di_hbm.at[pl.ds(g * GPB, GPB)], didx.at[0])
            pltpu.sync_copy(msg_hbm.at[pl.ds(g * GE, GE)], mb.at[0])
            for k in range(GPB):
                pltpu.sync_copy(mb.at[0, pl.ds(k * GB, GB)],
                                sh.at[didx.at[0, k]], add=True)

        plsc.subcore_barrier()

        @pl.when(sid < 10)
        def _():
            r = pl.ds(sid * 1000, 1000)
            pltpu.sync_copy(sh.at[r], acc_hbm.at[cid, r])

    return pl.kernel(
        body,
        out_type=[jax.ShapeDtypeStruct((NC, NN, CW), jnp.float32)],
        mesh=_sc_mesh(),
        scratch_types=[
            pltpu.VMEM((2, GPB, GB), jnp.int32),
            pltpu.VMEM((2, GE, CW), jnp.float32),
            pltpu.VMEM_SHARED((NN, CW), jnp.float32),
            pltpu.SemaphoreType.DMA,
            pltpu.SemaphoreType.DMA,
            pltpu.SemaphoreType.DMA,
            pltpu.SemaphoreType.DMA,
        ],
    )


_scatter_h = _make_scatter(HH)


def _edge_tc_body(gdc, gsc, ea,
                  w1a, w1b, w1c, w1d, b1, w2, b2,
                  wu, bu, wc1, bc1, wc2, bc2,
                  msg_o, eo_o):
    gd = gdc[:, :HID]
    gs = gsc[:, :HID]
    rel3 = gdc[:, HID:HID + 3] - gsc[:, HID:HID + 3]
    rd = jnp.sum(rel3 * rel3, axis=1, keepdims=True)
    eaf = ea[...].astype(jnp.float32)
    x1 = (gd @ w1a[...] + gs @ w1b[...] + eaf @ w1d[...]
          + rd * w1c[...] + b1[...])
    h1 = _silu(x1)
    mm = _silu(h1 @ w2[...] + b2[...])
    eo_o[...] = (mm @ wu[...] + bu[...] + eaf).astype(eo_o.dtype)
    c1 = _silu(mm @ wc1[...] + bc1[...])
    cw = jnp.sum(c1 * wc2[...], axis=1, keepdims=True) + bc2[...]
    pad = jnp.zeros((BE, CW - HID - 3), jnp.float32)
    msg_o[...] = jnp.concatenate([mm, rel3 * cw, pad], axis=1)


def _edge_tc_body_last(gdc, gsc, ea,
                       w1a, w1b, w1c, w1d, b1, w2, b2,
                       msg_o):
    gd = gdc[:, :HID]
    gs = gsc[:, :HID]
    rel3 = gdc[:, HID:HID + 3] - gsc[:, HID:HID + 3]
    rd = jnp.sum(rel3 * rel3, axis=1, keepdims=True)
    eaf = ea[...].astype(jnp.float32)
    x1 = (gd @ w1a[...] + gs @ w1b[...] + eaf @ w1d[...]
          + rd * w1c[...] + b1[...])
    mm = _silu(_silu(x1) @ w2[...] + b2[...])
    pad = jnp.zeros((BE, CW - HID), jnp.float32)
    msg_o[...] = jnp.concatenate([mm, pad], axis=1)


def _eb(d):
    return pl.BlockSpec((BE, d), lambda i: (i, 0))


def _wb(shape):
    nd = len(shape)
    return pl.BlockSpec(shape, lambda i: (0,) * nd)


_EDGE_W_SPECS = [
    _wb((HID, HID)), _wb((HID, HID)), _wb((1, HID)), _wb((HID, HID)),
    _wb((1, HID)), _wb((HID, HID)), _wb((1, HID)),
]

_edge_h = pl.pallas_call(
    _edge_tc_body,
    grid=(HH // BE,),
    in_specs=[_eb(CW), _eb(CW), _eb(HID)]
             + _EDGE_W_SPECS
             + [_wb((HID, HID)), _wb((1, HID)), _wb((HID, HID)),
                _wb((1, HID)), _wb((1, HID)), _wb((1, 1))],
    out_specs=[_eb(CW), _eb(HID)],
    out_shape=[
        jax.ShapeDtypeStruct((HH, CW), jnp.float32),
        jax.ShapeDtypeStruct((HH, HID), jnp.bfloat16),
    ],
)

_edge_last_h = pl.pallas_call(
    _edge_tc_body_last,
    grid=(HH // BE,),
    in_specs=[_eb(CW), _eb(CW), _eb(HID)] + _EDGE_W_SPECS,
    out_specs=[_eb(CW)],
    out_shape=[jax.ShapeDtypeStruct((HH, CW), jnp.float32)],
)


_INV_CNT = np.float32(1.0 / (NN * HID))


def _node_core(tab, msum, sc, sh, wn1a, wn1b, bn1, wn2, bn2,
               g_, be_, wf1, bf1, wf2, bf2):
    f0 = tab[:, :HID]
    m_i = msum[:, :HID]
    nh = _silu(f0 @ wn1a[...] + m_i @ wn1b[...] + bn1[...])
    nh = nh @ wn2[...] + bn2[...] + f0
    f = nh * (sc[...] + 1.0) + sh[...]
    mean = jnp.sum(f) * _INV_CNT
    xc = f - mean
    var = jnp.sum(xc * xc) * _INV_CNT
    fn = xc * lax.rsqrt(var + np.float32(1e-5)) * g_[...] + be_[...]
    fh = _gelu_exact(fn @ wf1[...] + bf1[...])
    return fh @ wf2[...] + bf2[...] + fn


def _node_tc_body(tab_r, acc0_r, acc1_r, sc, sh,
                  wn1a, wn1b, bn1, wn2, bn2, g_, be_, wf1, bf1, wf2, bf2,
                  tab_o):
    tab = tab_r[...]
    msum = acc0_r[0] + acc0_r[1] + acc1_r[0] + acc1_r[1]
    fnew = _node_core(tab, msum, sc, sh, wn1a, wn1b, bn1, wn2, bn2,
                      g_, be_, wf1, bf1, wf2, bf2)
    pos = tab[:, HID:HID + 16] + msum[:, HID:HID + 16]
    pad = jnp.zeros((NN, CW - HID - 16), jnp.float32)
    tab_o[...] = jnp.concatenate([fnew, pos, pad], axis=1)


def _node_tc_body_last(tab_r, acc0_r, acc1_r, sc, sh,
                       wn1a, wn1b, bn1, wn2, bn2, g_, be_, wf1, bf1, wf2, bf2,
                       wlin, blin, out_o):
    msum = acc0_r[0] + acc0_r[1] + acc1_r[0] + acc1_r[1]
    f = _node_core(tab_r[...], msum, sc, sh, wn1a, wn1b, bn1, wn2, bn2,
                   g_, be_, wf1, bf1, wf2, bf2)
    out_o[...] = f @ wlin[...] + blin[...]


_node_call = pl.pallas_call(
    _node_tc_body,
    out_shape=[jax.ShapeDtypeStruct((NN, CW), jnp.float32)],
)

_node_last_call = pl.pallas_call(
    _node_tc_body_last,
    out_shape=[jax.ShapeDtypeStruct((NN, OUTD), jnp.float32)],
)


def _time_tc_body(tval, freqs, wtm1, btm1, wtm2, btm2, wt, bt, temb_o):
    e = tval[...] * freqs[...]
    emb = jnp.concatenate([jnp.sin(e), jnp.cos(e)], axis=1)
    t1 = _silu(emb @ wtm1[...] + btm1[...])
    t2 = t1 @ wtm2[...] + btm2[...]
    st = _silu(t2)
    temb_o[...] = st @ wt[...] + bt[...]


_time_call = pl.pallas_call(
    _time_tc_body,
    out_shape=[jax.ShapeDtypeStruct((1, 2 * HID * NL), jnp.float32)],
)


def _row(b):
    return b.reshape(1, -1)


def kernel(x, pos, extra_x, edge_attr, ss, time, params, edge_index, batch):
    del ss, batch
    tab = jnp.concatenate(
        [x, extra_x, pos, jnp.zeros((NN, CW - 2 * 32 - 3), jnp.float32)], axis=1)
    si = [edge_index[0, h * HH:(h + 1) * HH].reshape(HH // GB, GB)
          for h in range(2)]
    di = [edge_index[1, h * HH:(h + 1) * HH].reshape(HH // GB, GB)
          for h in range(2)]
    ea = [edge_attr[:HH], edge_attr[HH:]]
    zc = jnp.zeros((NN, CW), jnp.float32)

    half = HID // 2
    freqs = jnp.exp(
        jnp.arange(half, dtype=jnp.float32)
        * np.float32(-math.log(10000.0) / (half - 1))).reshape(1, half)
    tm1, tm2 = params["time_mlp"]
    wt = jnp.concatenate([l["time"]["w"] for l in params["layers"]], axis=1)
    bt = jnp.concatenate([l["time"]["b"] for l in params["layers"]]).reshape(1, -1)
    (temb,) = _time_call(time.reshape(1, 1), freqs,
                         tm1["w"], _row(tm1["b"]), tm2["w"], _row(tm2["b"]),
                         wt, bt)

    out = None
    for l, lay in enumerate(params["layers"]):
        w1 = lay["edge_mlp"][0]["w"]
        ew = (w1[:HID], w1[HID:2 * HID], w1[2 * HID:2 * HID + 1],
              w1[2 * HID + 1:], _row(lay["edge_mlp"][0]["b"]),
              lay["edge_mlp"][1]["w"], _row(lay["edge_mlp"][1]["b"]))
        euw = (lay["edge_upd"]["w"], _row(lay["edge_upd"]["b"]),
               lay["coors_mlp"][0]["w"], _row(lay["coors_mlp"][0]["b"]),
               lay["coors_mlp"][1]["w"].reshape(1, HID),
               lay["coors_mlp"][1]["b"].reshape(1, 1))
        sc = temb[:, 2 * HID * l: 2 * HID * l + HID]
        sh = temb[:, 2 * HID * l + HID: 2 * HID * (l + 1)]
        nw1 = lay["node_mlp"][0]["w"]
        nws = (nw1[:HID], nw1[HID:], _row(lay["node_mlp"][0]["b"]),
               lay["node_mlp"][1]["w"], _row(lay["node_mlp"][1]["b"]),
               _row(lay["ff_norm"]["g"]), _row(lay["ff_norm"]["be"]),
               lay["ff"][0]["w"], _row(lay["ff"][0]["b"]),
               lay["ff"][1]["w"], _row(lay["ff"][1]["b"]))

        gd0, gs0 = _gather_h(tab, di[0], si[0])
        gd1, gs1 = _gather_h(tab, di[1], si[1])
        if l < NL - 1:
            msg0, eo0 = _edge_h(gd0, gs0, ea[0], *ew, *euw)
            (acc0,) = _scatter_h(msg0, di[0], zc)
            msg1, eo1 = _edge_h(gd1, gs1, ea[1], *ew, *euw)
            (acc1,) = _scatter_h(msg1, di[1], zc)
            (tab,) = _node_call(tab, acc0, acc1, sc, sh, *nws)
            ea = [eo0, eo1]
        else:
            (msg0,) = _edge_last_h(gd0, gs0, ea[0], *ew)
            (acc0,) = _scatter_h(msg0, di[0], zc)
            (msg1,) = _edge_last_h(gd1, gs1, ea[1], *ew)
            (acc1,) = _scatter_h(msg1, di[1], zc)
            (out,) = _node_last_call(tab, acc0, acc1, sc, sh, *nws,
                                     params["lin"]["w"],
                                     _row(params["lin"]["b"]))
    return out


# speedup vs baseline: 1.1867x; 1.0005x over previous
import math

import jax
import jax.numpy as jnp
import numpy as np
from jax import lax
from jax.experimental import pallas as pl
from jax.experimental.pallas import tpu as pltpu
from jax.experimental.pallas import tpu_sc as plsc

NN = 10000
EE = 320000
NCH = 2
HH = EE // NCH
HID = 64
NL = 4
OUTD = 20
CW = 128

NC, NS = 2, 16
NW = NC * NS
GB = 128

GPB = 1
GE = GB * GPB

BE = 2000


def _silu(x):
    return x / (1.0 + jnp.exp(-x))


def _gelu_exact(x):
    return 0.5 * x * (1.0 + lax.erf(x * np.float32(1.0 / math.sqrt(2.0))))


def _sc_mesh():
    return plsc.VectorSubcoreMesh(
        core_axis_name="c", subcore_axis_name="s",
        num_cores=NC, num_subcores=NS)


def _partition(wid, base, extra):
    ng = base + jnp.where(wid < extra, 1, 0)
    g0 = wid * base + jnp.minimum(wid, extra)
    return g0, ng


def _make_gather(ne):
    nblk = ne // GB
    base, extra = nblk // NW, nblk % NW

    def body(tab_hbm, di_hbm, si_hbm, gd_hbm, gs_hbm,
             didx, sidx, gdb, gsb,
             semi0, semi1, semg0, semg1, semw0, semw1):
        cid = lax.axis_index("c")
        sid = lax.axis_index("s")
        wid = sid * NC + cid
        g0, ng = _partition(wid, base, extra)
        nmain = (ng // 2) * 2
        g_last = g0 + nmain - 1
        semi = (semi0, semi1)
        semg = (semg0, semg1)
        semw = (semw0, semw1)

        def fire_idx(g, s, sem):
            pltpu.async_copy(di_hbm.at[g], didx.at[s], sem)
            pltpu.async_copy(si_hbm.at[g], sidx.at[s], sem)

        def drain_idx(s, sem):
            pltpu.make_async_copy(di_hbm.at[0], didx.at[s], sem).wait()
            pltpu.make_async_copy(si_hbm.at[0], sidx.at[s], sem).wait()

        def fire_gather(s, sem):
            pltpu.async_copy(tab_hbm.at[didx.at[s]], gdb.at[s], sem)
            pltpu.async_copy(tab_hbm.at[sidx.at[s]], gsb.at[s], sem)

        def drain_gather(s, sem):
            pltpu.make_async_copy(gd_hbm.at[pl.ds(0, GB)], gdb.at[s], sem).wait()
            pltpu.make_async_copy(gd_hbm.at[pl.ds(0, GB)], gsb.at[s], sem).wait()

        def fire_write(g, s, sem):
            eb = g * GB
            pltpu.async_copy(gdb.at[s], gd_hbm.at[pl.ds(eb, GB)], sem)
            pltpu.async_copy(gsb.at[s], gs_hbm.at[pl.ds(eb, GB)], sem)

        def drain_write(s, sem):
            pltpu.make_async_copy(gdb.at[s], gd_hbm.at[pl.ds(0, GB)], sem).wait()
            pltpu.make_async_copy(gsb.at[s], gs_hbm.at[pl.ds(0, GB)], sem).wait()

        fire_idx(g0, 0, semi[0])
        fire_idx(g0 + 1, 1, semi[1])

        def loop(j, carry):
            for k in range(2):
                g = g0 + j * 2 + k
                o = 1 - k

                @pl.when(g - 1 >= g0)
                def _():
                    drain_gather(o, semg[o])
                    fire_write(g - 1, o, semw[o])

                @pl.when((g - 1 >= g0) & (g + 1 <= g_last))
                def _():
                    fire_idx(g + 1, o, semi[o])

                @pl.when(g - 2 >= g0)
                def _():
                    drain_write(k, semw[k])

                drain_idx(k, semi[k])
                fire_gather(k, semg[k])
            return carry

        lax.fori_loop(0, ng // 2, loop, 0)

        drain_gather(1, semg[1])
        fire_write(g_last, 1, semw[1])
        drain_write(0, semw[0])
        drain_write(1, semw[1])

        @pl.when(ng > nmain)
        def _():
            g = g0 + nmain
            pltpu.sync_copy(di_hbm.at[g], didx.at[0])
            pltpu.sync_copy(si_hbm.at[g], sidx.at[0])
            pltpu.sync_copy(tab_hbm.at[didx.at[0]], gdb.at[0])
            pltpu.sync_copy(tab_hbm.at[sidx.at[0]], gsb.at[0])
            pltpu.sync_copy(gdb.at[0], gd_hbm.at[pl.ds(g * GB, GB)])
            pltpu.sync_copy(gsb.at[0], gs_hbm.at[pl.ds(g * GB, GB)])

    return pl.kernel(
        body,
        out_type=[
            jax.ShapeDtypeStruct((ne, CW), jnp.float32),
            jax.ShapeDtypeStruct((ne, CW), jnp.float32),
        ],
        mesh=_sc_mesh(),
        scratch_types=[
            pltpu.VMEM((2, GB), jnp.int32),
            pltpu.VMEM((2, GB), jnp.int32),
            pltpu.VMEM((2, GB, CW), jnp.float32),
            pltpu.VMEM((2, GB, CW), jnp.float32),
            pltpu.SemaphoreType.DMA,
            pltpu.SemaphoreType.DMA,
            pltpu.SemaphoreType.DMA,
            pltpu.SemaphoreType.DMA,
            pltpu.SemaphoreType.DMA,
            pltpu.SemaphoreType.DMA,
        ],
    )


_gather_h = _make_gather(HH)


def _make_scatter(ne):
    ngrp = ne // GE
    base, extra = ngrp // NW, ngrp % NW

    def body(msg_hbm, di_hbm, z_hbm, acc_hbm, didx, mb, sh,
             semp0, semp1, sema0, sema1):
        cid = lax.axis_index("c")
        sid = lax.axis_index("s")
        wid = sid * NC + cid
        g0, ng = _partition(wid, base, extra)
        nmain = (ng // 2) * 2
        g_last = g0 + nmain - 1
        semp = (semp0, semp1)
        sema = (sema0, sema1)

        @pl.when(sid == 0)
        def _():
            pltpu.sync_copy(z_hbm, sh)

        plsc.subcore_barrier()

        def fire_pf(g, s):
            pltpu.async_copy(di_hbm.at[pl.ds(g * GPB, GPB)], didx.at[s], semp[s])
            pltpu.async_copy(msg_hbm.at[pl.ds(g * GE, GE)], mb.at[s], semp[s])

        def drain_pf(s):
            pltpu.make_async_copy(
                di_hbm.at[pl.ds(0, GPB)], didx.at[s], semp[s]).wait()
            pltpu.make_async_copy(
                msg_hbm.at[pl.ds(0, GE)], mb.at[s], semp[s]).wait()

        def fire_adds(s):
            for k in range(GPB):
                pltpu.async_copy(mb.at[s, pl.ds(k * GB, GB)],
                                 sh.at[didx.at[s, k]], sema[s], add=True)

        def drain_adds(s):
            for k in range(GPB):
                pltpu.make_async_copy(mb.at[s, pl.ds(k * GB, GB)],
                                      sh.at[pl.ds(0, GB)], sema[s]).wait()

        fire_pf(g0, 0)

        def loop(j, carry):
            for k in range(2):
                g = g0 + j * 2 + k
                o = 1 - k
                drain_pf(k)

                @pl.when(g - 1 >= g0)
                def _():
                    drain_adds(o)

                @pl.when(g + 1 <= g_last)
                def _():
                    fire_pf(g + 1, o)

                fire_adds(k)
            return carry

        lax.fori_loop(0, ng // 2, loop, 0)
        drain_adds(1)

        @pl.when(ng > nmain)
        def _():
            g = g0 + nmain
            pltpu.sync_copy(di_hbm.at[pl.ds(g * GPB, GPB)], didx.at[0])
            pltpu.sync_copy(msg_hbm.at[pl.ds(g * GE, GE)], mb.at[0])
            for k in range(GPB):
                pltpu.sync_copy(mb.at[0, pl.ds(k * GB, GB)],
                                sh.at[didx.at[0, k]], add=True)

        plsc.subcore_barrier()

        @pl.when(sid < 10)
        def _():
            r = pl.ds(sid * 1000, 1000)
            pltpu.sync_copy(sh.at[r], acc_hbm.at[cid, r])

    return pl.kernel(
        body,
        out_type=[jax.ShapeDtypeStruct((NC, NN, CW), jnp.float32)],
        mesh=_sc_mesh(),
        scratch_types=[
            pltpu.VMEM((2, GPB, GB), jnp.int32),
            pltpu.VMEM((2, GE, CW), jnp.float32),
            pltpu.VMEM_SHARED((NN, CW), jnp.float32),
            pltpu.SemaphoreType.DMA,
            pltpu.SemaphoreType.DMA,
            pltpu.SemaphoreType.DMA,
            pltpu.SemaphoreType.DMA,
        ],
    )


_scatter_h = _make_scatter(HH)


def _edge_tc_body(gdc, gsc, ea,
                  w1a, w1b, w1c, w1d, b1, w2, b2,
                  wu, bu, wc1, bc1, wc2, bc2,
                  msg_o, eo_o):
    gd = gdc[:, :HID]
    gs = gsc[:, :HID]
    rel3 = gdc[:, HID:HID + 3] - gsc[:, HID:HID + 3]
    rd = jnp.sum(rel3 * rel3, axis=1, keepdims=True)
    eaf = ea[...].astype(jnp.float32)
    x1 = (gd @ w1a[...] + gs @ w1b[...] + eaf @ w1d[...]
          + rd * w1c[...] + b1[...])
    h1 = _silu(x1)
    mm = _silu(h1 @ w2[...] + b2[...])
    eo_o[...] = (mm @ wu[...] + bu[...] + eaf).astype(eo_o.dtype)
    c1 = _silu(mm @ wc1[...] + bc1[...])
    cw = jnp.sum(c1 * wc2[...], axis=1, keepdims=True) + bc2[...]
    pad = jnp.zeros((BE, CW - HID - 3), jnp.float32)
    msg_o[...] = jnp.concatenate([mm, rel3 * cw, pad], axis=1)


def _edge_tc_body_last(gdc, gsc, ea,
                       w1a, w1b, w1c, w1d, b1, w2, b2,
                       msg_o):
    gd = gdc[:, :HID]
    gs = gsc[:, :HID]
    rel3 = gdc[:, HID:HID + 3] - gsc[:, HID:HID + 3]
    rd = jnp.sum(rel3 * rel3, axis=1, keepdims=True)
    eaf = ea[...].astype(jnp.float32)
    x1 = (gd @ w1a[...] + gs @ w1b[...] + eaf @ w1d[...]
          + rd * w1c[...] + b1[...])
    mm = _silu(_silu(x1) @ w2[...] + b2[...])
    pad = jnp.zeros((BE, CW - HID), jnp.float32)
    msg_o[...] = jnp.concatenate([mm, pad], axis=1)


def _eb(d):
    return pl.BlockSpec((BE, d), lambda i: (i, 0))


def _wb(shape):
    nd = len(shape)
    return pl.BlockSpec(shape, lambda i: (0,) * nd)


_EDGE_W_SPECS = [
    _wb((HID, HID)), _wb((HID, HID)), _wb((1, HID)), _wb((HID, HID)),
    _wb((1, HID)), _wb((HID, HID)), _wb((1, HID)),
]

_edge_h = pl.pallas_call(
    _edge_tc_body,
    grid=(HH // BE,),
    in_specs=[_eb(CW), _eb(CW), _eb(HID)]
             + _EDGE_W_SPECS
             + [_wb((HID, HID)), _wb((1, HID)), _wb((HID, HID)),
                _wb((1, HID)), _wb((1, HID)), _wb((1, 1))],
    out_specs=[_eb(CW), _eb(HID)],
    out_shape=[
        jax.ShapeDtypeStruct((HH, CW), jnp.float32),
        jax.ShapeDtypeStruct((HH, HID), jnp.bfloat16),
    ],
)

_edge_last_h = pl.pallas_call(
    _edge_tc_body_last,
    grid=(HH // BE,),
    in_specs=[_eb(CW), _eb(CW), _eb(HID)] + _EDGE_W_SPECS,
    out_specs=[_eb(CW)],
    out_shape=[jax.ShapeDtypeStruct((HH, CW), jnp.float32)],
)


_INV_CNT = np.float32(1.0 / (NN * HID))


def _node_core(tab, msum, sc, sh, wn1a, wn1b, bn1, wn2, bn2,
               g_, be_, wf1, bf1, wf2, bf2):
    f0 = tab[:, :HID]
    m_i = msum[:, :HID]
    nh = _silu(f0 @ wn1a[...] + m_i @ wn1b[...] + bn1[...])
    nh = nh @ wn2[...] + bn2[...] + f0
    f = nh * (sc[...] + 1.0) + sh[...]
    mean = jnp.sum(f) * _INV_CNT
    xc = f - mean
    var = jnp.sum(xc * xc) * _INV_CNT
    fn = xc * lax.rsqrt(var + np.float32(1e-5)) * g_[...] + be_[...]
    fh = _gelu_exact(fn @ wf1[...] + bf1[...])
    return fh @ wf2[...] + bf2[...] + fn


def _node_tc_body(tab_r, a0, a1, sc, sh,
                  wn1a, wn1b, bn1, wn2, bn2, g_, be_, wf1, bf1, wf2, bf2,
                  tab_o):
    tab = tab_r[...]
    msum = (a0[0] + a0[1]) + (a1[0] + a1[1])
    fnew = _node_core(tab, msum, sc, sh, wn1a, wn1b, bn1, wn2, bn2,
                      g_, be_, wf1, bf1, wf2, bf2)
    pos = tab[:, HID:HID + 16] + msum[:, HID:HID + 16]
    pad = jnp.zeros((NN, CW - HID - 16), jnp.float32)
    tab_o[...] = jnp.concatenate([fnew, pos, pad], axis=1)


def _node_tc_body_last(tab_r, a0, a1, sc, sh,
                       wn1a, wn1b, bn1, wn2, bn2, g_, be_, wf1, bf1, wf2, bf2,
                       wlin, blin, out_o):
    msum = (a0[0] + a0[1]) + (a1[0] + a1[1])
    f = _node_core(tab_r[...], msum, sc, sh, wn1a, wn1b, bn1, wn2, bn2,
                   g_, be_, wf1, bf1, wf2, bf2)
    out_o[...] = f @ wlin[...] + blin[...]


_node_call = pl.pallas_call(
    _node_tc_body,
    out_shape=[jax.ShapeDtypeStruct((NN, CW), jnp.float32)],
)

_node_last_call = pl.pallas_call(
    _node_tc_body_last,
    out_shape=[jax.ShapeDtypeStruct((NN, OUTD), jnp.float32)],
)


def _time_tc_body(tval, freqs, wtm1, btm1, wtm2, btm2, wt, bt, temb_o):
    e = tval[...] * freqs[...]
    emb = jnp.concatenate([jnp.sin(e), jnp.cos(e)], axis=1)
    t1 = _silu(emb @ wtm1[...] + btm1[...])
    t2 = t1 @ wtm2[...] + btm2[...]
    st = _silu(t2)
    temb_o[...] = st @ wt[...] + bt[...]


_time_call = pl.pallas_call(
    _time_tc_body,
    out_shape=[jax.ShapeDtypeStruct((1, 2 * HID * NL), jnp.float32)],
)


def _row(b):
    return b.reshape(1, -1)


def kernel(x, pos, extra_x, edge_attr, ss, time, params, edge_index, batch):
    del ss, batch
    tab = jnp.concatenate(
        [x, extra_x, pos, jnp.zeros((NN, CW - 2 * 32 - 3), jnp.float32)], axis=1)
    si = [edge_index[0, h * HH:(h + 1) * HH].reshape(HH // GB, GB)
          for h in range(NCH)]
    di = [edge_index[1, h * HH:(h + 1) * HH].reshape(HH // GB, GB)
          for h in range(NCH)]
    ea = [edge_attr[h * HH:(h + 1) * HH] for h in range(NCH)]
    zc = jnp.zeros((NN, CW), jnp.float32)

    half = HID // 2
    freqs = jnp.exp(
        jnp.arange(half, dtype=jnp.float32)
        * np.float32(-math.log(10000.0) / (half - 1))).reshape(1, half)
    tm1, tm2 = params["time_mlp"]
    wt = jnp.concatenate([l["time"]["w"] for l in params["layers"]], axis=1)
    bt = jnp.concatenate([l["time"]["b"] for l in params["layers"]]).reshape(1, -1)
    (temb,) = _time_call(time.reshape(1, 1), freqs,
                         tm1["w"], _row(tm1["b"]), tm2["w"], _row(tm2["b"]),
                         wt, bt)

    out = None
    for l, lay in enumerate(params["layers"]):
        w1 = lay["edge_mlp"][0]["w"]
        ew = (w1[:HID], w1[HID:2 * HID], w1[2 * HID:2 * HID + 1],
              w1[2 * HID + 1:], _row(lay["edge_mlp"][0]["b"]),
              lay["edge_mlp"][1]["w"], _row(lay["edge_mlp"][1]["b"]))
        euw = (lay["edge_upd"]["w"], _row(lay["edge_upd"]["b"]),
               lay["coors_mlp"][0]["w"], _row(lay["coors_mlp"][0]["b"]),
               lay["coors_mlp"][1]["w"].reshape(1, HID),
               lay["coors_mlp"][1]["b"].reshape(1, 1))
        sc = temb[:, 2 * HID * l: 2 * HID * l + HID]
        sh = temb[:, 2 * HID * l + HID: 2 * HID * (l + 1)]
        nw1 = lay["node_mlp"][0]["w"]
        nws = (nw1[:HID], nw1[HID:], _row(lay["node_mlp"][0]["b"]),
               lay["node_mlp"][1]["w"], _row(lay["node_mlp"][1]["b"]),
               _row(lay["ff_norm"]["g"]), _row(lay["ff_norm"]["be"]),
               lay["ff"][0]["w"], _row(lay["ff"][0]["b"]),
               lay["ff"][1]["w"], _row(lay["ff"][1]["b"]))

        g = [_gather_h(tab, di[h], si[h]) for h in range(NCH)]
        accs = []
        if l < NL - 1:
            eos = []
            for h in range(NCH):
                msg, eo = _edge_h(g[h][0], g[h][1], ea[h], *ew, *euw)
                eos.append(eo)
                accs.append(_scatter_h(msg, di[h], zc)[0])
            (tab,) = _node_call(tab, *accs, sc, sh, *nws)
            ea = eos
        else:
            for h in range(NCH):
                (msg,) = _edge_last_h(g[h][0], g[h][1], ea[h], *ew)
                accs.append(_scatter_h(msg, di[h], zc)[0])
            (out,) = _node_last_call(tab, *accs, sc, sh, *nws,
                                     params["lin"]["w"],
                                     _row(params["lin"]["b"]))
    return out
